# baseline (speedup 1.0000x reference)
import functools

import jax
import jax.numpy as jnp
from jax import lax
from jax.experimental import pallas as pl
from jax.experimental.pallas import tpu as pltpu
from jax.experimental.pallas import tpu_sc as plsc

B, S, MAXC = 1024, 200, 16
DW, DC, DOUT = 128, 64, 128
VC = 256
N = B * S
NC, NS = 2, 16
NW = NC * NS
TPW = N // NW
C = 128
NCHUNK = TPW // C
VEC = 16
NGRP = DOUT // VEC


def _proj_f32_body(t_ref, w_ref, b_ref, out_ref):
    out_ref[...] = (
        jnp.dot(t_ref[...], w_ref[...], preferred_element_type=jnp.float32)
        + b_ref[...]
    )


def _project(table, w, b2d, bm):
    m, k = table.shape
    return pl.pallas_call(
        _proj_f32_body,
        grid=(m // bm,),
        in_specs=[
            pl.BlockSpec((bm, k), lambda i: (i, 0)),
            pl.BlockSpec((k, DOUT), lambda i: (0, 0)),
            pl.BlockSpec((1, DOUT), lambda i: (0, 0)),
        ],
        out_specs=pl.BlockSpec((bm, DOUT), lambda i: (i, 0)),
        out_shape=jax.ShapeDtypeStruct((m, DOUT), jnp.float32),
    )(table, w, b2d)


def _chunk_compute(pc_v, cid_v, len_v, rows_v, offs):
    def grp_body(tg, carry2):
        t0 = tg * VEC
        lens = len_v[pl.ds(t0, VEC)]
        for k in range(VEC):
            t = t0 + k
            nchars = lens[k]
            cvec = cid_v[t, :]
            accs = tuple(
                rows_v[t, pl.ds(VEC * c, VEC)] for c in range(NGRP))

            def char_body(j, a):
                rv = cvec.at[jnp.full((VEC,), 0, jnp.int32) + j].get(
                    mode="promise_in_bounds")
                return tuple(
                    a[c] + plsc.load_gather(pc_v, [rv, offs[c]])
                    for c in range(NGRP))

            accs = lax.fori_loop(0, nchars, char_body, accs)
            for c in range(NGRP):
                rows_v[t, pl.ds(VEC * c, VEC)] = accs[c]
        return carry2

    lax.fori_loop(0, C // VEC, grp_body, 0)


@functools.partial(
    pl.kernel,
    out_type=jax.ShapeDtypeStruct((N, DOUT), jnp.float32),
    mesh=plsc.VectorSubcoreMesh(core_axis_name="c", subcore_axis_name="s"),
    scratch_types=[
        pltpu.VMEM((VC, DOUT), jnp.float32),
        pltpu.VMEM((C,), jnp.int32),
        pltpu.VMEM((C,), jnp.int32),
        pltpu.VMEM((C, MAXC), jnp.int32),
        pltpu.VMEM((C,), jnp.int32),
        pltpu.VMEM((C, DOUT), jnp.float32),
        pltpu.VMEM((C, DOUT), jnp.float32),
        pltpu.SemaphoreType.DMA,
        pltpu.SemaphoreType.DMA,
        pltpu.SemaphoreType.DMA,
        pltpu.SemaphoreType.DMA,
    ],
    compiler_params=pltpu.CompilerParams(needs_layout_passes=False),
)
def _sc_embed(pw_hbm, pc_hbm, widx_hbm, cid_hbm, len_hbm, out_hbm,
              pc_v, idx0, idx1, cid_v, len_v,
              rows0, rows1, sem0, sem1, semw0, semw1):
    wid = lax.axis_index("s") * NC + lax.axis_index("c")
    base0 = wid * TPW
    pltpu.sync_copy(pc_hbm, pc_v)
    lane = lax.iota(jnp.int32, VEC)
    offs = [lane + VEC * c for c in range(NGRP)]

    def pair_body(i, carry):
        ba = base0 + 2 * i * C
        bb = ba + C
        pltpu.sync_copy(widx_hbm.at[pl.ds(ba, C)], idx0)
        h0 = pltpu.async_copy(pw_hbm.at[idx0], rows0, sem0)
        pltpu.sync_copy(widx_hbm.at[pl.ds(bb, C)], idx1)
        h1 = pltpu.async_copy(pw_hbm.at[idx1], rows1, sem1)
        pltpu.sync_copy(cid_hbm.at[pl.ds(ba, C)], cid_v)
        pltpu.sync_copy(len_hbm.at[pl.ds(ba, C)], len_v)
        h0.wait()
        _chunk_compute(pc_v, cid_v, len_v, rows0, offs)
        w0 = pltpu.async_copy(rows0, out_hbm.at[pl.ds(ba, C)], semw0)
        pltpu.sync_copy(cid_hbm.at[pl.ds(bb, C)], cid_v)
        pltpu.sync_copy(len_hbm.at[pl.ds(bb, C)], len_v)
        h1.wait()
        _chunk_compute(pc_v, cid_v, len_v, rows1, offs)
        w1 = pltpu.async_copy(rows1, out_hbm.at[pl.ds(bb, C)], semw1)
        w0.wait()
        w1.wait()
        return carry

    lax.fori_loop(0, NCHUNK // 2, pair_body, 0)


def kernel(word_inputs, char_ids, char_lengths, word_table, char_table, W, b):
    pw = _project(word_table, W[:DW], b.reshape(1, DOUT), 1000)
    pc = _project(char_table, W[DW:], jnp.zeros((1, DOUT), jnp.float32), 256)
    widx = word_inputs.reshape(N).astype(jnp.int32)
    cid = char_ids.reshape(N, MAXC).astype(jnp.int32)
    clen = char_lengths.reshape(N).astype(jnp.int32)
    out = _sc_embed(pw, pc, widx, cid, clen)
    return out.reshape(B, S, DOUT)
